# Initial kernel scaffold; baseline (speedup 1.0000x reference)
import jax
import jax.numpy as jnp
from jax.experimental import pallas as pl


def kernel(x, root_w, root_b, sons_w, sons_b):
    raise NotImplementedError("write your pallas kernel here")


# speedup vs baseline: 3.5272x; 3.5272x over previous
import functools

import jax
import jax.numpy as jnp
from jax.experimental import pallas as pl

B = 2048
D = 768
E = 8
TOP_K = 2
BT = 256


def _fused_moe_kernel(x_ref, rw_ref, rb_ref, sw_ref, sb_ref, out_ref):
    x = x_ref[...]
    logits = jnp.dot(x, rw_ref[...], preferred_element_type=jnp.float32)
    logits = logits + rb_ref[...][None, :]

    idx = jax.lax.broadcasted_iota(jnp.int32, (BT, E), 1)
    neg = jnp.float32(-1.7e38)

    v1 = jnp.max(logits, axis=1, keepdims=True)
    i1 = jnp.min(jnp.where(logits == v1, idx, E), axis=1, keepdims=True)
    masked = jnp.where(idx == i1, neg, logits)
    v2 = jnp.max(masked, axis=1, keepdims=True)
    i2 = jnp.min(jnp.where(masked == v2, idx, E), axis=1, keepdims=True)

    w1 = 1.0 / (1.0 + jnp.exp(v2 - v1))
    w2 = 1.0 - w1
    wmat = w1 * (idx == i1).astype(jnp.float32) + w2 * (idx == i2).astype(
        jnp.float32
    )

    acc = jnp.dot(wmat, sb_ref[...], preferred_element_type=jnp.float32)
    for e in range(E):
        y = jnp.dot(x, sw_ref[e], preferred_element_type=jnp.float32)
        acc = acc + wmat[:, e : e + 1] * y
    out_ref[...] = acc


@jax.jit
def kernel(x, root_w, root_b, sons_w, sons_b):
    out = pl.pallas_call(
        _fused_moe_kernel,
        grid=(B // BT,),
        in_specs=[
            pl.BlockSpec((BT, D), lambda i: (i, 0)),
            pl.BlockSpec((D, E), lambda i: (0, 0)),
            pl.BlockSpec((E,), lambda i: (0,)),
            pl.BlockSpec((E, D, D), lambda i: (0, 0, 0)),
            pl.BlockSpec((E, D), lambda i: (0, 0)),
        ],
        out_specs=pl.BlockSpec((BT, D), lambda i: (i, 0)),
        out_shape=jax.ShapeDtypeStruct((B, D), jnp.float32),
    )(x, root_w, root_b, sons_w, sons_b)
    return out[:, None, :]
